# baseline (speedup 1.0000x reference)
import jax
import jax.numpy as jnp
from jax.experimental import pallas as pl
from jax.experimental.pallas import tpu as pltpu

TAU_ = 1.0
C_ = 100
CPAD_ = 128
B_ = 2048
D_ = 1024
BLK_ = 256
NB_ = B_ // BLK_

_DN_ROWS = (((0,), (0,)), ((), ()))
_DN_FEAT = (((1,), (1,)), ((), ()))


def _loss_kernel(z1_ref, z2_ref, lab_ref, out_ref,
                 zb1_s, zb2_s, inv1_s, inv2_s, sums_s):
    f32 = jnp.float32
    bf16 = jnp.bfloat16
    j = pl.program_id(0)

    lab_blk = lab_ref[pl.ds(j * BLK_, BLK_), :]
    col = jax.lax.broadcasted_iota(jnp.int32, (BLK_, CPAD_), 1)
    oh_blk = (lab_blk == col).astype(f32)

    def prep(z_ref, zb_s, inv_s):
        z = z_ref[...]
        ss = jnp.sum(z * z, axis=1, keepdims=True)
        inv = jax.lax.rsqrt(jnp.maximum(ss, 1e-24))
        zb = z.astype(bf16)
        zb_s[pl.ds(j * BLK_, BLK_), :] = zb
        inv_s[pl.ds(j * BLK_, BLK_), :] = inv
        ohs = (oh_blk * inv).astype(bf16)
        return jax.lax.dot_general(ohs, zb, _DN_ROWS,
                                   preferred_element_type=f32)

    part = (prep(z1_ref, zb1_s, inv1_s) + prep(z2_ref, zb2_s, inv2_s))

    @pl.when(j == 0)
    def _first():
        sums_s[...] = part

    @pl.when(j > 0)
    def _acc():
        sums_s[...] += part

    @pl.when(j == NB_ - 1)
    def _phase1():
        lab = lab_ref[...]
        colf = jax.lax.broadcasted_iota(jnp.int32, (B_, CPAD_), 1)
        oh = (lab == colf).astype(f32)
        counts = 2.0 * jnp.sum(oh, axis=0, keepdims=True)
        invc = (1.0 / jnp.maximum(counts, 1.0)) * (1.0 / TAU_)
        sumsb = sums_s[...].astype(bf16)
        vmask = (jax.lax.broadcasted_iota(jnp.int32, (1, CPAD_), 1)
                 < C_).astype(f32)

        def view_loss(zb_s, inv_s):
            zb = zb_s[...]
            inv = inv_s[...]
            simr = jax.lax.dot_general(zb, sumsb, _DN_FEAT,
                                       preferred_element_type=f32)
            sim = simr * invc * inv
            p = jnp.sum(sim * oh, axis=1, keepdims=True)
            s = jnp.sum(jnp.exp(sim - p) * vmask, axis=1, keepdims=True)
            return jnp.log(s) - p

        total = jnp.sum(view_loss(zb1_s, inv1_s) + view_loss(zb2_s, inv2_s),
                        axis=0, keepdims=True)
        out_ref[...] = total * (1.0 / (2.0 * B_))


def kernel(z1_features, z2_features, labels):
    lab2d = labels.astype(jnp.int32).reshape(B_, 1)
    out = pl.pallas_call(
        _loss_kernel,
        grid=(NB_,),
        in_specs=[
            pl.BlockSpec((BLK_, D_), lambda j: (j, 0)),
            pl.BlockSpec((BLK_, D_), lambda j: (j, 0)),
            pl.BlockSpec((B_, 1), lambda j: (0, 0)),
        ],
        out_specs=pl.BlockSpec((1, 1), lambda j: (0, 0)),
        out_shape=jax.ShapeDtypeStruct((1, 1), jnp.float32),
        scratch_shapes=[
            pltpu.VMEM((B_, D_), jnp.bfloat16),
            pltpu.VMEM((B_, D_), jnp.bfloat16),
            pltpu.VMEM((B_, 1), jnp.float32),
            pltpu.VMEM((B_, 1), jnp.float32),
            pltpu.VMEM((CPAD_, D_), jnp.float32),
        ],
        compiler_params=pltpu.CompilerParams(
            dimension_semantics=("arbitrary",),
            vmem_limit_bytes=100 * 1024 * 1024,
        ),
    )(z1_features, z2_features, lab2d)
    return out[0, 0]


# speedup vs baseline: 1.0894x; 1.0894x over previous
import jax
import jax.numpy as jnp
from jax.experimental import pallas as pl
from jax.experimental.pallas import tpu as pltpu

TAU_ = 1.0
C_ = 100
CPAD_ = 128
B_ = 2048
D_ = 1024
BLK_ = 256
NB_ = B_ // BLK_
NSLOT_ = 3

_DN_ROWS = (((0,), (0,)), ((), ()))
_DN_FEAT = (((1,), (1,)), ((), ()))


def _copy(z_hbm, buf, sem, blk, slot):
    return pltpu.make_async_copy(
        z_hbm.at[pl.ds(blk * BLK_, BLK_), :], buf.at[slot], sem.at[slot])


def _loss_kernel(z1_hbm, z2_hbm, lab_ref, out_ref,
                 buf1, buf2, zb1_s, zb2_s, inv1_s, inv2_s, sums_s,
                 sem1, sem2):
    f32 = jnp.float32
    bf16 = jnp.bfloat16
    j = pl.program_id(0)
    slot = jax.lax.rem(j, NSLOT_)

    @pl.when(j == 0)
    def _prologue():
        _copy(z1_hbm, buf1, sem1, 0, 0).start()
        _copy(z2_hbm, buf2, sem2, 0, 0).start()
        _copy(z1_hbm, buf1, sem1, 1, 1).start()
        _copy(z2_hbm, buf2, sem2, 1, 1).start()

    @pl.when(j + 2 < NB_)
    def _prefetch():
        nslot = jax.lax.rem(j + 2, NSLOT_)
        _copy(z1_hbm, buf1, sem1, j + 2, nslot).start()
        _copy(z2_hbm, buf2, sem2, j + 2, nslot).start()

    _copy(z1_hbm, buf1, sem1, j, slot).wait()
    _copy(z2_hbm, buf2, sem2, j, slot).wait()

    lab_blk = lab_ref[pl.ds(j * BLK_, BLK_), :]
    col = jax.lax.broadcasted_iota(jnp.int32, (BLK_, CPAD_), 1)
    oh_blk = (lab_blk == col).astype(f32)

    def prep(buf, zb_s, inv_s):
        z = buf[slot]
        ss = jnp.sum(z * z, axis=1, keepdims=True)
        inv = jax.lax.rsqrt(jnp.maximum(ss, 1e-24))
        zb = z.astype(bf16)
        zb_s[pl.ds(j * BLK_, BLK_), :] = zb
        inv_s[pl.ds(j * BLK_, BLK_), :] = inv
        ohs = (oh_blk * inv).astype(bf16)
        return jax.lax.dot_general(ohs, zb, _DN_ROWS,
                                   preferred_element_type=f32)

    part = prep(buf1, zb1_s, inv1_s) + prep(buf2, zb2_s, inv2_s)

    @pl.when(j == 0)
    def _first():
        sums_s[...] = part

    @pl.when(j > 0)
    def _acc():
        sums_s[...] += part

    @pl.when(j == NB_ - 1)
    def _phase1():
        lab = lab_ref[...]
        colf = jax.lax.broadcasted_iota(jnp.int32, (B_, CPAD_), 1)
        oh = (lab == colf).astype(f32)
        counts = 2.0 * jnp.sum(oh, axis=0, keepdims=True)
        invc = (1.0 / jnp.maximum(counts, 1.0)) * (1.0 / TAU_)
        sumsb = sums_s[...].astype(bf16)
        vmask = (jax.lax.broadcasted_iota(jnp.int32, (1, CPAD_), 1)
                 < C_).astype(f32)

        def view_loss(zb_s, inv_s):
            zb = zb_s[...]
            inv = inv_s[...]
            simr = jax.lax.dot_general(zb, sumsb, _DN_FEAT,
                                       preferred_element_type=f32)
            sim = simr * invc * inv
            p = jnp.sum(sim * oh, axis=1, keepdims=True)
            s = jnp.sum(jnp.exp(sim - p) * vmask, axis=1, keepdims=True)
            return jnp.log(s) - p

        total = jnp.sum(view_loss(zb1_s, inv1_s) + view_loss(zb2_s, inv2_s),
                        axis=0, keepdims=True)
        out_ref[...] = total * (1.0 / (2.0 * B_))


def kernel(z1_features, z2_features, labels):
    lab2d = labels.astype(jnp.int32).reshape(B_, 1)
    out = pl.pallas_call(
        _loss_kernel,
        grid=(NB_,),
        in_specs=[
            pl.BlockSpec(memory_space=pltpu.MemorySpace.HBM),
            pl.BlockSpec(memory_space=pltpu.MemorySpace.HBM),
            pl.BlockSpec((B_, 1), lambda j: (0, 0)),
        ],
        out_specs=pl.BlockSpec((1, 1), lambda j: (0, 0)),
        out_shape=jax.ShapeDtypeStruct((1, 1), jnp.float32),
        scratch_shapes=[
            pltpu.VMEM((NSLOT_, BLK_, D_), jnp.float32),
            pltpu.VMEM((NSLOT_, BLK_, D_), jnp.float32),
            pltpu.VMEM((B_, D_), jnp.bfloat16),
            pltpu.VMEM((B_, D_), jnp.bfloat16),
            pltpu.VMEM((B_, 1), jnp.float32),
            pltpu.VMEM((B_, 1), jnp.float32),
            pltpu.VMEM((CPAD_, D_), jnp.float32),
            pltpu.SemaphoreType.DMA((NSLOT_,)),
            pltpu.SemaphoreType.DMA((NSLOT_,)),
        ],
        compiler_params=pltpu.CompilerParams(
            dimension_semantics=("arbitrary",),
            vmem_limit_bytes=100 * 1024 * 1024,
        ),
    )(z1_features, z2_features, lab2d)
    return out[0, 0]


# speedup vs baseline: 1.1681x; 1.0723x over previous
import jax
import jax.numpy as jnp
from jax.experimental import pallas as pl
from jax.experimental.pallas import tpu as pltpu

TAU_ = 1.0
C_ = 100
CPAD_ = 128
B_ = 2048
D_ = 1024
BLK_ = 512
NB_ = B_ // BLK_
NSLOT_ = 3

_DN_ROWS = (((0,), (0,)), ((), ()))
_DN_FEAT = (((1,), (1,)), ((), ()))


def _copy(z_hbm, buf, sem, blk, slot):
    return pltpu.make_async_copy(
        z_hbm.at[pl.ds(blk * BLK_, BLK_), :], buf.at[slot], sem.at[slot])


def _loss_kernel(z1_hbm, z2_hbm, lab_ref, out_ref,
                 buf1, buf2, zb1_s, zb2_s, inv1_s, inv2_s, sums_s,
                 sem1, sem2):
    f32 = jnp.float32
    bf16 = jnp.bfloat16
    j = pl.program_id(0)
    slot = jax.lax.rem(j, NSLOT_)

    @pl.when(j == 0)
    def _prologue():
        _copy(z1_hbm, buf1, sem1, 0, 0).start()
        _copy(z2_hbm, buf2, sem2, 0, 0).start()
        _copy(z1_hbm, buf1, sem1, 1, 1).start()
        _copy(z2_hbm, buf2, sem2, 1, 1).start()

    @pl.when(j + 2 < NB_)
    def _prefetch():
        nslot = jax.lax.rem(j + 2, NSLOT_)
        _copy(z1_hbm, buf1, sem1, j + 2, nslot).start()
        _copy(z2_hbm, buf2, sem2, j + 2, nslot).start()

    _copy(z1_hbm, buf1, sem1, j, slot).wait()
    _copy(z2_hbm, buf2, sem2, j, slot).wait()

    lab_blk = lab_ref[pl.ds(j * BLK_, BLK_), :]
    col = jax.lax.broadcasted_iota(jnp.int32, (BLK_, CPAD_), 1)
    oh_blk = (lab_blk == col).astype(f32)

    def prep(buf, zb_s, inv_s):
        z = buf[slot]
        ss = jnp.sum(z * z, axis=1, keepdims=True)
        inv = jax.lax.rsqrt(jnp.maximum(ss, 1e-24))
        zb = z.astype(bf16)
        zb_s[pl.ds(j * BLK_, BLK_), :] = zb
        inv_s[pl.ds(j * BLK_, BLK_), :] = inv
        ohs = (oh_blk * inv).astype(bf16)
        return jax.lax.dot_general(ohs, zb, _DN_ROWS,
                                   preferred_element_type=f32)

    part = prep(buf1, zb1_s, inv1_s) + prep(buf2, zb2_s, inv2_s)

    @pl.when(j == 0)
    def _first():
        sums_s[...] = part

    @pl.when(j > 0)
    def _acc():
        sums_s[...] += part

    @pl.when(j == NB_ - 1)
    def _phase1():
        lab = lab_ref[...]
        colf = jax.lax.broadcasted_iota(jnp.int32, (B_, CPAD_), 1)
        oh = (lab == colf).astype(f32)
        counts = 2.0 * jnp.sum(oh, axis=0, keepdims=True)
        invc = (1.0 / jnp.maximum(counts, 1.0)) * (1.0 / TAU_)
        sumsb = sums_s[...].astype(bf16)
        vmask = (jax.lax.broadcasted_iota(jnp.int32, (1, CPAD_), 1)
                 < C_).astype(f32)

        def view_loss(zb_s, inv_s):
            zb = zb_s[...]
            inv = inv_s[...]
            simr = jax.lax.dot_general(zb, sumsb, _DN_FEAT,
                                       preferred_element_type=f32)
            sim = simr * invc * inv
            p = jnp.sum(sim * oh, axis=1, keepdims=True)
            s = jnp.sum(jnp.exp(sim - p) * vmask, axis=1, keepdims=True)
            return jnp.log(s) - p

        total = jnp.sum(view_loss(zb1_s, inv1_s) + view_loss(zb2_s, inv2_s),
                        axis=0, keepdims=True)
        out_ref[...] = total * (1.0 / (2.0 * B_))


def kernel(z1_features, z2_features, labels):
    lab2d = labels.astype(jnp.int32).reshape(B_, 1)
    out = pl.pallas_call(
        _loss_kernel,
        grid=(NB_,),
        in_specs=[
            pl.BlockSpec(memory_space=pltpu.MemorySpace.HBM),
            pl.BlockSpec(memory_space=pltpu.MemorySpace.HBM),
            pl.BlockSpec((B_, 1), lambda j: (0, 0)),
        ],
        out_specs=pl.BlockSpec((1, 1), lambda j: (0, 0)),
        out_shape=jax.ShapeDtypeStruct((1, 1), jnp.float32),
        scratch_shapes=[
            pltpu.VMEM((NSLOT_, BLK_, D_), jnp.float32),
            pltpu.VMEM((NSLOT_, BLK_, D_), jnp.float32),
            pltpu.VMEM((B_, D_), jnp.bfloat16),
            pltpu.VMEM((B_, D_), jnp.bfloat16),
            pltpu.VMEM((B_, 1), jnp.float32),
            pltpu.VMEM((B_, 1), jnp.float32),
            pltpu.VMEM((CPAD_, D_), jnp.float32),
            pltpu.SemaphoreType.DMA((NSLOT_,)),
            pltpu.SemaphoreType.DMA((NSLOT_,)),
        ],
        compiler_params=pltpu.CompilerParams(
            dimension_semantics=("arbitrary",),
            vmem_limit_bytes=100 * 1024 * 1024,
        ),
    )(z1_features, z2_features, lab2d)
    return out[0, 0]
